# baseline (speedup 1.0000x reference)
import functools

import jax
import jax.numpy as jnp
from jax import lax
from jax.experimental import pallas as pl
from jax.experimental.pallas import tpu as pltpu
from jax.experimental.pallas import tpu_sc as plsc


def _pack_body(l_ref, u_ref, o_ref):
    l = l_ref[...]
    u = u_ref[...]
    cl = 1.0 - jnp.minimum(l, u)
    cu = 1.0 - jnp.maximum(l, u)
    bl = lax.bitcast_convert_type(cl, jnp.int32)
    bu = lax.bitcast_convert_type(cu, jnp.int32)
    rl = (bl + 0x7FFF + ((bl >> 16) & 1)) >> 16
    ru = (bu + 0x7FFF + ((bu >> 16) & 1)) >> 16
    o_ref[...] = (ru << 16) | (rl & 0xFFFF)


def _combine_body(parts_ref, pb_ref, l_ref, u_ref):
    n = pb_ref.shape[0]
    al = parts_ref[0, pl.ds(0, n)] + parts_ref[3, pl.ds(0, n)]
    au = parts_ref[1, pl.ds(0, n)] + parts_ref[4, pl.ds(0, n)]
    dg = parts_ref[2, pl.ds(0, n)] + parts_ref[5, pl.ds(0, n)]
    den = jnp.maximum(dg, 1e-6)
    pb = pb_ref[...]
    l_ref[...] = jnp.clip(pb - al / den, 0.0, 1.0)
    u_ref[...] = jnp.clip(pb - au / den, 0.0, 1.0)


def kernel(L0, U0, pW, pB, edge_index):
    N = pB.shape[0]
    E = edge_index.shape[1]
    M = pW.shape[0]
    NW = 32
    NPAD = 102400
    ZS = NPAD // 16
    CE = 800
    edges_per_tile = E // NW
    nchunks = edges_per_tile // CE

    f32 = jnp.float32
    zeros = jnp.zeros((NPAD,), f32)
    pad = NPAD - N
    l0p = jnp.pad(L0, (0, pad))
    u0p = jnp.pad(U0, (0, pad))

    packed = pl.pallas_call(
        _pack_body,
        out_shape=jax.ShapeDtypeStruct((NPAD,), jnp.int32),
    )(l0p, u0p)

    mesh = plsc.VectorSubcoreMesh(core_axis_name="c", subcore_axis_name="s",
                                  num_cores=2, num_subcores=16)
    ibuf2 = [pltpu.VMEM((CE,), jnp.int32)] * 2
    fbuf2 = [pltpu.VMEM((CE,), f32)] * 2

    @functools.partial(
        pl.kernel,
        out_type=jax.ShapeDtypeStruct((6, NPAD), f32),
        mesh=mesh,
        compiler_params=pltpu.CompilerParams(use_tc_tiling_on_sc=False,
                                             needs_layout_passes=False),
        scratch_types=[
            pltpu.VMEM((N,), jnp.int32),
            ibuf2,
            ibuf2,
            fbuf2,
            fbuf2,
            fbuf2,
            pltpu.VMEM((CE,), f32),
            pltpu.VMEM_SHARED((NPAD,), f32),
            pltpu.VMEM_SHARED((NPAD,), f32),
            pltpu.VMEM_SHARED((NPAD,), f32),
            pltpu.SemaphoreType.DMA,
            pltpu.SemaphoreType.DMA,
        ],
    )
    def _edge_kernel(edge_hbm, pw_hbm, packed_hbm, zeros_hbm, out_hbm,
                     table_v, src_v, dst_v, pw_v, ml_v, mu_v, ones_v,
                     accl_s, accu_s, accw_s, sem_in, sem_sc):
        cid = lax.axis_index("c")
        sid = lax.axis_index("s")
        wid = sid * 2 + cid
        zoff = sid * ZS
        for acc in (accl_s, accu_s, accw_s):
            pltpu.sync_copy(zeros_hbm.at[pl.ds(zoff, ZS)],
                            acc.at[pl.ds(zoff, ZS)])
        pltpu.sync_copy(packed_hbm.at[pl.ds(0, N)], table_v)

        @pl.loop(0, CE // 16)
        def _init_ones(i):
            ones_v[pl.ds(i * 16, 16)] = jnp.full((16,), 1.0, f32)

        plsc.subcore_barrier()

        base = wid * edges_per_tile

        def fire_inputs(k, p, weighted, method="start"):
            eb = base + k * CE
            for hbm, v in ((edge_hbm.at[0], src_v), (edge_hbm.at[1], dst_v)) + (
                    ((pw_hbm, pw_v),) if weighted else ()):
                d = pltpu.make_async_copy(hbm.at[pl.ds(eb, CE)], v[p], sem_in)
                getattr(d, method)()

        def scatter(p, weighted, method="start"):
            wsrc = pw_v[p] if weighted else ones_v
            for v, acc in ((ml_v[p], accl_s), (mu_v[p], accu_s),
                           (wsrc, accw_s)):
                d = pltpu.make_async_copy(v, acc.at[dst_v[p]], sem_sc)
                d.start(add=True) if method == "start" else d.wait()

        def compute(p, weighted):
            @pl.loop(0, CE // 16, unroll=4)
            def _row(i):
                sl = pl.ds(i * 16, 16)
                g = plsc.load_gather(table_v, [src_v[p][sl]])
                cl = plsc.bitcast(g << 16, f32)
                cu = plsc.bitcast(g & jnp.int32(-65536), f32)
                if weighted:
                    w = pw_v[p][sl]
                    ml_v[p][sl] = w * cl
                    mu_v[p][sl] = w * cu
                else:
                    ml_v[p][sl] = cl
                    mu_v[p][sl] = cu

        def run_chunks(weighted):
            fire_inputs(0, 0, weighted)

            @pl.loop(0, nchunks // 2)
            def _chunk2(k2):
                for p in (0, 1):
                    k = 2 * k2 + p
                    fire_inputs(k, p, weighted, "wait")
                    compute(p, weighted)
                    if p == 0:
                        @pl.when(k2 > 0)
                        def _():
                            scatter(1, weighted, "wait")
                    else:
                        scatter(0, weighted, "wait")
                    scatter(p, weighted)
                    if p == 0:
                        fire_inputs(k + 1, 1, weighted)
                    else:
                        @pl.when(k2 < nchunks // 2 - 1)
                        def _():
                            fire_inputs(2 * k2 + 2, 0, weighted)

            scatter(1, weighted, "wait")

        tile_weighted = base < M

        @pl.when(tile_weighted)
        def _():
            run_chunks(True)

        @pl.when(jnp.logical_not(tile_weighted))
        def _():
            run_chunks(False)

        plsc.subcore_barrier()
        for j, acc in enumerate((accl_s, accu_s, accw_s)):
            pltpu.sync_copy(acc.at[pl.ds(zoff, ZS)],
                            out_hbm.at[cid * 3 + j, pl.ds(zoff, ZS)])

    parts = _edge_kernel(edge_index, pW, packed, zeros)

    lp, up = pl.pallas_call(
        _combine_body,
        out_shape=[jax.ShapeDtypeStruct((N,), f32)] * 2,
    )(parts, pB)
    return lp, up


# speedup vs baseline: 1.1558x; 1.1558x over previous
import functools

import jax
import jax.numpy as jnp
from jax import lax
from jax.experimental import pallas as pl
from jax.experimental.pallas import tpu as pltpu
from jax.experimental.pallas import tpu_sc as plsc


def _pack_body(l_ref, u_ref, o_ref):
    l = l_ref[...]
    u = u_ref[...]
    cl = 1.0 - jnp.minimum(l, u)
    cu = 1.0 - jnp.maximum(l, u)
    bl = lax.bitcast_convert_type(cl, jnp.int32)
    bu = lax.bitcast_convert_type(cu, jnp.int32)
    rl = (bl + 0x7FFF + ((bl >> 16) & 1)) >> 16
    ru = (bu + 0x7FFF + ((bu >> 16) & 1)) >> 16
    o_ref[...] = (ru << 16) | (rl & 0xFFFF)


def _combine_body(parts_ref, pb_ref, l_ref, u_ref):
    n = pb_ref.shape[0]
    al = parts_ref[0, pl.ds(0, n)] + parts_ref[3, pl.ds(0, n)]
    au = parts_ref[1, pl.ds(0, n)] + parts_ref[4, pl.ds(0, n)]
    dg = parts_ref[2, pl.ds(0, n)] + parts_ref[5, pl.ds(0, n)]
    den = jnp.maximum(dg, 1e-6)
    pb = pb_ref[...]
    l_ref[...] = jnp.clip(pb - al / den, 0.0, 1.0)
    u_ref[...] = jnp.clip(pb - au / den, 0.0, 1.0)


def kernel(L0, U0, pW, pB, edge_index):
    N = pB.shape[0]
    E = edge_index.shape[1]
    M = pW.shape[0]
    NW = 32
    NPAD = 102400
    ZS = NPAD // 16
    CE = 800
    edges_per_tile = E // NW
    nchunks = edges_per_tile // CE

    f32 = jnp.float32
    ei_flat = edge_index.reshape(2 * E)
    zeros = jnp.zeros((NPAD,), f32)
    pad = NPAD - N
    l0p = jnp.pad(L0, (0, pad))
    u0p = jnp.pad(U0, (0, pad))

    packed = pl.pallas_call(
        _pack_body,
        out_shape=jax.ShapeDtypeStruct((NPAD,), jnp.int32),
    )(l0p, u0p)

    mesh = plsc.VectorSubcoreMesh(core_axis_name="c", subcore_axis_name="s",
                                  num_cores=2, num_subcores=16)
    ibuf2 = [pltpu.VMEM((CE,), jnp.int32)] * 2
    fbuf2 = [pltpu.VMEM((CE,), f32)] * 2

    @functools.partial(
        pl.kernel,
        out_type=jax.ShapeDtypeStruct((6, NPAD), f32),
        mesh=mesh,
        compiler_params=pltpu.CompilerParams(use_tc_tiling_on_sc=False,
                                             needs_layout_passes=False),
        scratch_types=[
            pltpu.VMEM((N,), jnp.int32),
            ibuf2,
            ibuf2,
            fbuf2,
            fbuf2,
            fbuf2,
            pltpu.VMEM((CE,), f32),
            pltpu.VMEM_SHARED((NPAD,), f32),
            pltpu.VMEM_SHARED((NPAD,), f32),
            pltpu.VMEM_SHARED((NPAD,), f32),
            pltpu.SemaphoreType.DMA,
            pltpu.SemaphoreType.DMA,
        ],
    )
    def _edge_kernel(ei_hbm, pw_hbm, packed_hbm, zeros_hbm, out_hbm,
                     table_v, src_v, dst_v, pw_v, ml_v, mu_v, ones_v,
                     accl_s, accu_s, accw_s, sem_in, sem_sc):
        cid = lax.axis_index("c")
        sid = lax.axis_index("s")
        wid = sid * 2 + cid
        zoff = sid * ZS
        for acc in (accl_s, accu_s, accw_s):
            pltpu.sync_copy(zeros_hbm.at[pl.ds(zoff, ZS)],
                            acc.at[pl.ds(zoff, ZS)])
        pltpu.sync_copy(packed_hbm.at[pl.ds(0, N)], table_v)

        @pl.loop(0, CE // 16)
        def _init_ones(i):
            ones_v[pl.ds(i * 16, 16)] = jnp.full((16,), 1.0, f32)

        plsc.subcore_barrier()

        base = wid * edges_per_tile

        def fire_inputs(k, p, weighted, method="start"):
            eb = base + k * CE
            for off, hbm, v in ((0, ei_hbm, src_v), (E, ei_hbm, dst_v)) + (
                    ((None, pw_hbm, pw_v),) if weighted else ()):
                sl = pl.ds(eb if off is None else off + eb, CE)
                d = pltpu.make_async_copy(hbm.at[sl], v[p], sem_in)
                getattr(d, method)()

        def scatter(p, weighted, method="start"):
            wsrc = pw_v[p] if weighted else ones_v
            for v, acc in ((ml_v[p], accl_s), (mu_v[p], accu_s),
                           (wsrc, accw_s)):
                d = pltpu.make_async_copy(v, acc.at[dst_v[p]], sem_sc)
                d.start(add=True) if method == "start" else d.wait()

        def compute(p, weighted):
            @pl.loop(0, CE // 16)
            def _row(i):
                sl = pl.ds(i * 16, 16)
                g = plsc.load_gather(table_v, [src_v[p][sl]])
                cl = plsc.bitcast(g << 16, f32)
                cu = plsc.bitcast(g & jnp.int32(-65536), f32)
                if weighted:
                    w = pw_v[p][sl]
                    ml_v[p][sl] = w * cl
                    mu_v[p][sl] = w * cu
                else:
                    ml_v[p][sl] = cl
                    mu_v[p][sl] = cu

        def run_chunks(weighted):
            fire_inputs(0, 0, weighted)

            @pl.loop(0, nchunks // 2)
            def _chunk2(k2):
                for p in (0, 1):
                    k = 2 * k2 + p
                    fire_inputs(k, p, weighted, "wait")
                    compute(p, weighted)
                    if p == 0:
                        @pl.when(k2 > 0)
                        def _():
                            scatter(1, weighted, "wait")
                    else:
                        scatter(0, weighted, "wait")
                    scatter(p, weighted)
                    if p == 0:
                        fire_inputs(k + 1, 1, weighted)
                    else:
                        @pl.when(k2 < nchunks // 2 - 1)
                        def _():
                            fire_inputs(2 * k2 + 2, 0, weighted)

            scatter(1, weighted, "wait")

        tile_weighted = base < M

        @pl.when(tile_weighted)
        def _():
            run_chunks(True)

        @pl.when(jnp.logical_not(tile_weighted))
        def _():
            run_chunks(False)

        plsc.subcore_barrier()
        for j, acc in enumerate((accl_s, accu_s, accw_s)):
            pltpu.sync_copy(acc.at[pl.ds(zoff, ZS)],
                            out_hbm.at[cid * 3 + j, pl.ds(zoff, ZS)])

    parts = _edge_kernel(ei_flat, pW, packed, zeros)

    lp, up = pl.pallas_call(
        _combine_body,
        out_shape=[jax.ShapeDtypeStruct((N,), f32)] * 2,
    )(parts, pB)
    return lp, up


# speedup vs baseline: 1.1608x; 1.0043x over previous
import functools

import jax
import jax.numpy as jnp
from jax import lax
from jax.experimental import pallas as pl
from jax.experimental.pallas import tpu as pltpu
from jax.experimental.pallas import tpu_sc as plsc


def _pack_body(l_ref, u_ref, o_ref):
    l = l_ref[...]
    u = u_ref[...]
    cl = 1.0 - jnp.minimum(l, u)
    cu = 1.0 - jnp.maximum(l, u)
    bl = lax.bitcast_convert_type(cl, jnp.int32)
    bu = lax.bitcast_convert_type(cu, jnp.int32)
    rl = (bl + 0x7FFF + ((bl >> 16) & 1)) >> 16
    ru = (bu + 0x7FFF + ((bu >> 16) & 1)) >> 16
    o_ref[...] = (ru << 16) | (rl & 0xFFFF)


def _combine_body(parts_ref, pb_ref, l_ref, u_ref):
    n = pb_ref.shape[0]
    al = parts_ref[0, pl.ds(0, n)] + parts_ref[3, pl.ds(0, n)]
    au = parts_ref[1, pl.ds(0, n)] + parts_ref[4, pl.ds(0, n)]
    dg = parts_ref[2, pl.ds(0, n)] + parts_ref[5, pl.ds(0, n)]
    den = jnp.maximum(dg, 1e-6)
    pb = pb_ref[...]
    l_ref[...] = jnp.clip(pb - al / den, 0.0, 1.0)
    u_ref[...] = jnp.clip(pb - au / den, 0.0, 1.0)


def kernel(L0, U0, pW, pB, edge_index):
    N = pB.shape[0]
    E = edge_index.shape[1]
    M = pW.shape[0]
    NW = 32
    NPAD = 102400
    ZS = NPAD // 16
    CE = 800
    edges_per_tile = E // NW
    nchunks = edges_per_tile // CE

    f32 = jnp.float32
    ei_flat = edge_index.reshape(2 * E)
    zeros = jnp.zeros((NPAD,), f32)
    pad = NPAD - N
    l0p = jnp.pad(L0, (0, pad))
    u0p = jnp.pad(U0, (0, pad))

    packed = pl.pallas_call(
        _pack_body,
        out_shape=jax.ShapeDtypeStruct((NPAD,), jnp.int32),
    )(l0p, u0p)

    mesh = plsc.VectorSubcoreMesh(core_axis_name="c", subcore_axis_name="s",
                                  num_cores=2, num_subcores=16)
    ibuf2 = [pltpu.VMEM((CE,), jnp.int32)] * 2
    fbuf2 = [pltpu.VMEM((CE,), f32)] * 2

    @functools.partial(
        pl.kernel,
        out_type=jax.ShapeDtypeStruct((6, NPAD), f32),
        mesh=mesh,
        compiler_params=pltpu.CompilerParams(use_tc_tiling_on_sc=False,
                                             needs_layout_passes=False),
        scratch_types=[
            pltpu.VMEM((N,), jnp.int32),
            ibuf2,
            ibuf2,
            fbuf2,
            fbuf2,
            fbuf2,
            pltpu.VMEM((CE,), f32),
            pltpu.VMEM_SHARED((NPAD,), f32),
            pltpu.VMEM_SHARED((NPAD,), f32),
            pltpu.VMEM_SHARED((NPAD,), f32),
            pltpu.SemaphoreType.DMA,
            [pltpu.SemaphoreType.DMA] * 2,
        ],
    )
    def _edge_kernel(ei_hbm, pw_hbm, packed_hbm, zeros_hbm, out_hbm,
                     table_v, src_v, dst_v, pw_v, ml_v, mu_v, ones_v,
                     accl_s, accu_s, accw_s, sem_in, sem_sc):
        cid = lax.axis_index("c")
        sid = lax.axis_index("s")
        wid = sid * 2 + cid
        zoff = sid * ZS
        tdma = pltpu.make_async_copy(packed_hbm.at[pl.ds(0, N)], table_v,
                                     sem_in)
        tdma.start()
        for acc in (accl_s, accu_s, accw_s):
            pltpu.sync_copy(zeros_hbm.at[pl.ds(zoff, ZS)],
                            acc.at[pl.ds(zoff, ZS)])

        @pl.loop(0, CE // 16)
        def _init_ones(i):
            ones_v[pl.ds(i * 16, 16)] = jnp.full((16,), 1.0, f32)

        tdma.wait()
        plsc.subcore_barrier()

        base = wid * edges_per_tile

        def fire_inputs(k, p, weighted, method="start"):
            eb = base + k * CE
            for off, hbm, v in ((0, ei_hbm, src_v), (E, ei_hbm, dst_v)) + (
                    ((None, pw_hbm, pw_v),) if weighted else ()):
                sl = pl.ds(eb if off is None else off + eb, CE)
                d = pltpu.make_async_copy(hbm.at[sl], v[p], sem_in)
                getattr(d, method)()

        def scatter(p, weighted, method="start"):
            wsrc = pw_v[p] if weighted else ones_v
            for v, acc in ((ml_v[p], accl_s), (mu_v[p], accu_s),
                           (wsrc, accw_s)):
                d = pltpu.make_async_copy(v, acc.at[dst_v[p]], sem_sc[p])
                d.start(add=True) if method == "start" else d.wait()

        def compute(p, weighted):
            @pl.loop(0, CE // 16)
            def _row(i):
                sl = pl.ds(i * 16, 16)
                g = plsc.load_gather(table_v, [src_v[p][sl]])
                cl = plsc.bitcast(g << 16, f32)
                cu = plsc.bitcast(g & jnp.int32(-65536), f32)
                if weighted:
                    w = pw_v[p][sl]
                    ml_v[p][sl] = w * cl
                    mu_v[p][sl] = w * cu
                else:
                    ml_v[p][sl] = cl
                    mu_v[p][sl] = cu

        def run_chunks(weighted):
            fire_inputs(0, 0, weighted)

            @pl.loop(0, nchunks // 2)
            def _chunk2(k2):
                for p in (0, 1):
                    k = 2 * k2 + p
                    fire_inputs(k, p, weighted, "wait")
                    compute(p, weighted)
                    scatter(p, weighted)
                    if p == 0:
                        @pl.when(k2 > 0)
                        def _():
                            scatter(1, weighted, "wait")
                    else:
                        scatter(0, weighted, "wait")
                    if p == 0:
                        fire_inputs(k + 1, 1, weighted)
                    else:
                        @pl.when(k2 < nchunks // 2 - 1)
                        def _():
                            fire_inputs(2 * k2 + 2, 0, weighted)

            scatter(1, weighted, "wait")

        tile_weighted = base < M

        @pl.when(tile_weighted)
        def _():
            run_chunks(True)

        @pl.when(jnp.logical_not(tile_weighted))
        def _():
            run_chunks(False)

        plsc.subcore_barrier()
        for j, acc in enumerate((accl_s, accu_s, accw_s)):
            pltpu.sync_copy(acc.at[pl.ds(zoff, ZS)],
                            out_hbm.at[cid * 3 + j, pl.ds(zoff, ZS)])

    parts = _edge_kernel(ei_flat, pW, packed, zeros)

    lp, up = pl.pallas_call(
        _combine_body,
        out_shape=[jax.ShapeDtypeStruct((N,), f32)] * 2,
    )(parts, pB)
    return lp, up
